# baseline (speedup 1.0000x reference)
import functools

import jax
import jax.numpy as jnp
from jax import lax
from jax.experimental import pallas as pl
from jax.experimental.pallas import tpu as pltpu
from jax.experimental.pallas import tpu_sc as plsc

B = 16384
D = 64
L = 16

_info = plsc.get_sparse_core_info()
NC, NS = _info.num_cores, _info.num_subcores
NW = NC * NS
BPW = B // NW
NCH = 32
NCHUNK = BPW // NCH


def _sc_body(idx_hbm, w_hbm, v_hbm, u_hbm, out_hbm,
             idx_v, wg, vg, ug, out_v, sem):
    wid = lax.axis_index("s") * NC + lax.axis_index("c")

    pltpu.sync_copy(idx_hbm.at[wid], idx_v)

    lane = jnp.arange(L, dtype=jnp.int32)
    bufs = (wg, vg, ug)
    tabs = (w_hbm, v_hbm, u_hbm)

    def chunk_body(ci, carry):
        base = ci * NCH
        ivecs = [[idx_v[pl.ds(t * BPW + base + h * L, L)] for h in range(2)]
                 for t in range(3)]
        g8s = [[(iv >> 3) << 3 for iv in tvecs] for tvecs in ivecs]
        r8s = [[iv & 7 for iv in tvecs] for tvecs in ivecs]

        for h in range(2):
            for r in range(L):
                j = h * L + r
                for t in range(3):
                    g8 = pl.multiple_of(g8s[t][h][r], 8)
                    pltpu.async_copy(tabs[t].at[pl.ds(g8, 8), :],
                                     bufs[t].at[pl.ds(j * 8, 8), :], sem)
        for j in range(NCH):
            for t in range(3):
                pltpu.make_async_copy(tabs[t].at[pl.ds(0, 8), :],
                                      bufs[t].at[pl.ds(j * 8, 8), :],
                                      sem).wait()

        for h in range(2):
            thetas = jnp.zeros((L,), jnp.float32)
            for r in range(L):
                j = h * L + r
                rw = j * 8 + r8s[0][h][r]
                rv = j * 8 + r8s[1][h][r]
                ru = j * 8 + r8s[2][h][r]
                acc = jnp.zeros((L,), jnp.float32)
                for c in range(D // L):
                    sl = pl.ds(c * L, L)
                    acc = acc + wg[rw, sl] * vg[rv, sl] * ug[ru, sl]
                theta = jnp.sum(acc)
                thetas = thetas + jnp.where(lane == r, theta, 0.0)
            probs = 1.0 / (1.0 + jnp.exp(-thetas))
            out_v[pl.ds(base + h * L, L)] = probs
        return carry

    lax.fori_loop(0, NCHUNK, chunk_body, 0)

    pltpu.sync_copy(out_v, out_hbm.at[pl.ds(wid * BPW, BPW)])


@functools.partial(jax.jit, static_argnums=())
def kernel(indices, W, V, U):
    idx = indices.astype(jnp.int32).T
    idx = idx.reshape(3, NW, BPW).transpose(1, 0, 2).reshape(NW, 3 * BPW)

    mesh = plsc.VectorSubcoreMesh(core_axis_name="c", subcore_axis_name="s")
    run = pl.kernel(
        _sc_body,
        mesh=mesh,
        out_type=jax.ShapeDtypeStruct((B,), jnp.float32),
        scratch_types=[
            pltpu.VMEM((3 * BPW,), jnp.int32),
            pltpu.VMEM((NCH * 8, D), jnp.float32),
            pltpu.VMEM((NCH * 8, D), jnp.float32),
            pltpu.VMEM((NCH * 8, D), jnp.float32),
            pltpu.VMEM((BPW,), jnp.float32),
            pltpu.SemaphoreType.DMA,
        ],
        compiler_params=pltpu.CompilerParams(needs_layout_passes=False),
    )
    return run(idx, W, V, U)


# speedup vs baseline: 1.0236x; 1.0236x over previous
import functools

import jax
import jax.numpy as jnp
from jax import lax
from jax.experimental import pallas as pl
from jax.experimental.pallas import tpu as pltpu
from jax.experimental.pallas import tpu_sc as plsc

B = 16384
D = 64
L = 16

_info = plsc.get_sparse_core_info()
NC, NS = _info.num_cores, _info.num_subcores
NW = NC * NS
BPW = B // NW
NCH = L
NCHUNK = BPW // NCH


def _sc_body(idx_hbm, w_hbm, v_hbm, u_hbm, out_hbm,
             idx_v, wgA, vgA, ugA, wgB, vgB, ugB, out_v, semA, semB):
    wid = lax.axis_index("s") * NC + lax.axis_index("c")

    pltpu.sync_copy(idx_hbm.at[wid], idx_v)

    lane = jnp.arange(L, dtype=jnp.int32)
    tabs = (w_hbm, v_hbm, u_hbm)
    bufsA = (wgA, vgA, ugA)
    bufsB = (wgB, vgB, ugB)

    def fire(ci, bufs, sem):
        ivecs = [idx_v[pl.ds(t * BPW + ci * NCH, L)] for t in range(3)]
        g8s = [(iv >> 3) << 3 for iv in ivecs]
        for r in range(L):
            for t in range(3):
                g8 = pl.multiple_of(g8s[t][r], 8)
                pltpu.async_copy(tabs[t].at[pl.ds(g8, 8), :],
                                 bufs[t].at[pl.ds(r * 8, 8), :], sem)

    def drain(bufs, sem):
        for r in range(L):
            for t in range(3):
                pltpu.make_async_copy(tabs[t].at[pl.ds(0, 8), :],
                                      bufs[t].at[pl.ds(r * 8, 8), :],
                                      sem).wait()

    def compute(ci, bufs):
        wg, vg, ug = bufs
        ivecs = [idx_v[pl.ds(t * BPW + ci * NCH, L)] for t in range(3)]
        r8s = [iv & 7 for iv in ivecs]
        thetas = jnp.zeros((L,), jnp.float32)
        for r in range(L):
            rw = r * 8 + r8s[0][r]
            rv = r * 8 + r8s[1][r]
            ru = r * 8 + r8s[2][r]
            acc = jnp.zeros((L,), jnp.float32)
            for c in range(D // L):
                sl = pl.ds(c * L, L)
                acc = acc + wg[rw, sl] * vg[rv, sl] * ug[ru, sl]
            theta = jnp.sum(acc)
            thetas = thetas + jnp.where(lane == r, theta, 0.0)
        probs = 1.0 / (1.0 + jnp.exp(-thetas))
        out_v[pl.ds(ci * NCH, L)] = probs

    fire(0, bufsA, semA)
    fire(1, bufsB, semB)

    def body(m, carry):
        c0 = 2 * m
        drain(bufsA, semA)
        compute(c0, bufsA)
        fire(c0 + 2, bufsA, semA)
        drain(bufsB, semB)
        compute(c0 + 1, bufsB)
        fire(c0 + 3, bufsB, semB)
        return carry

    lax.fori_loop(0, NCHUNK // 2 - 1, body, 0)

    drain(bufsA, semA)
    compute(NCHUNK - 2, bufsA)
    drain(bufsB, semB)
    compute(NCHUNK - 1, bufsB)

    pltpu.sync_copy(out_v, out_hbm.at[pl.ds(wid * BPW, BPW)])


@functools.partial(jax.jit, static_argnums=())
def kernel(indices, W, V, U):
    idx = indices.astype(jnp.int32).T
    idx = idx.reshape(3, NW, BPW).transpose(1, 0, 2).reshape(NW, 3 * BPW)

    mesh = plsc.VectorSubcoreMesh(core_axis_name="c", subcore_axis_name="s")
    run = pl.kernel(
        _sc_body,
        mesh=mesh,
        out_type=jax.ShapeDtypeStruct((B,), jnp.float32),
        scratch_types=[
            pltpu.VMEM((3 * BPW,), jnp.int32),
            pltpu.VMEM((NCH * 8, D), jnp.float32),
            pltpu.VMEM((NCH * 8, D), jnp.float32),
            pltpu.VMEM((NCH * 8, D), jnp.float32),
            pltpu.VMEM((NCH * 8, D), jnp.float32),
            pltpu.VMEM((NCH * 8, D), jnp.float32),
            pltpu.VMEM((NCH * 8, D), jnp.float32),
            pltpu.VMEM((BPW,), jnp.float32),
            pltpu.SemaphoreType.DMA,
            pltpu.SemaphoreType.DMA,
        ],
        compiler_params=pltpu.CompilerParams(needs_layout_passes=False),
    )
    return run(idx, W, V, U)
